# baseline (speedup 1.0000x reference)
import jax
import jax.numpy as jnp
from jax.experimental import pallas as pl
from jax.experimental.pallas import tpu as pltpu

ALPHA = 20.0
BETA = 0.01

B = 128
D = 1024
T = 32768
BLOCK_T = 4096
NSTEPS = T // BLOCK_T


def _loss_kernel(orig_ref, masked_ref, masks_ref, out_ref, acc_ref):
    i = pl.program_id(0)

    @pl.when(i == 0)
    def _init():
        acc_ref[0, 0] = 0.0

    u = jnp.exp(-jnp.abs(masks_ref[...]))
    v = 1.0 + u
    acc_ref[0, 0] += jnp.sum(u / (v * v))

    @pl.when(i == NSTEPS - 1)
    def _finish():
        o = orig_ref[...]
        m = masked_ref[...]
        d = m - o
        mse = jnp.sum(d * d) * (1.0 / (B * D))
        na = jnp.maximum(jnp.sqrt(jnp.sum(o * o, axis=1)), 1e-8)
        nb = jnp.maximum(jnp.sqrt(jnp.sum(m * m, axis=1)), 1e-8)
        dot = jnp.sum(o * m, axis=1)
        cos = jnp.mean(1.0 - dot / (na * nb))
        semantic = mse + 0.1 * cos
        binary = acc_ref[0, 0] * (1.0 / (B * T))
        total = ALPHA * semantic + BETA * binary
        lane = jax.lax.broadcasted_iota(jnp.int32, (1, 128), 1)
        row = jnp.where(
            lane == 0, total,
            jnp.where(lane == 1, semantic,
                      jnp.where(lane == 2, binary, 0.0)))
        out_ref[...] = row


def kernel(orig_embeds, masked_embeds, masks):
    out = pl.pallas_call(
        _loss_kernel,
        grid=(NSTEPS,),
        in_specs=[
            pl.BlockSpec((B, D), lambda i: (0, 0)),
            pl.BlockSpec((B, D), lambda i: (0, 0)),
            pl.BlockSpec((B, BLOCK_T), lambda i: (0, i)),
        ],
        out_specs=pl.BlockSpec((1, 128), lambda i: (0, 0)),
        out_shape=jax.ShapeDtypeStruct((1, 128), jnp.float32),
        scratch_shapes=[pltpu.SMEM((1, 1), jnp.float32)],
        compiler_params=pltpu.CompilerParams(
            dimension_semantics=("arbitrary",),
        ),
    )(orig_embeds, masked_embeds, masks)
    return (out[0, 0], out[0, 1], out[0, 2])


# speedup vs baseline: 1.0577x; 1.0577x over previous
import jax
import jax.numpy as jnp
from jax.experimental import pallas as pl
from jax.experimental.pallas import tpu as pltpu

ALPHA = 20.0
BETA = 0.01

B = 128
D = 1024
T = 32768
BLOCK_T = 4096
NSTEPS = T // BLOCK_T


def _loss_kernel(orig_ref, masked_ref, masks_ref, out_ref, acc_ref):
    i = pl.program_id(0)

    @pl.when(i == 0)
    def _init():
        acc_ref[0, 0] = 0.0

    t = jnp.tanh(masks_ref[...] * 0.5)
    acc_ref[0, 0] += jnp.sum(t * t)

    @pl.when(i == NSTEPS - 1)
    def _finish():
        o = orig_ref[...]
        m = masked_ref[...]
        d = m - o
        mse = jnp.sum(d * d) * (1.0 / (B * D))
        na = jnp.maximum(jnp.sqrt(jnp.sum(o * o, axis=1)), 1e-8)
        nb = jnp.maximum(jnp.sqrt(jnp.sum(m * m, axis=1)), 1e-8)
        dot = jnp.sum(o * m, axis=1)
        cos = jnp.mean(1.0 - dot / (na * nb))
        semantic = mse + 0.1 * cos
        binary = 0.25 - 0.25 * acc_ref[0, 0] * (1.0 / (B * T))
        total = ALPHA * semantic + BETA * binary
        lane = jax.lax.broadcasted_iota(jnp.int32, (1, 128), 1)
        row = jnp.where(
            lane == 0, total,
            jnp.where(lane == 1, semantic,
                      jnp.where(lane == 2, binary, 0.0)))
        out_ref[...] = row


def kernel(orig_embeds, masked_embeds, masks):
    out = pl.pallas_call(
        _loss_kernel,
        grid=(NSTEPS,),
        in_specs=[
            pl.BlockSpec((B, D), lambda i: (0, 0)),
            pl.BlockSpec((B, D), lambda i: (0, 0)),
            pl.BlockSpec((B, BLOCK_T), lambda i: (0, i)),
        ],
        out_specs=pl.BlockSpec((1, 128), lambda i: (0, 0)),
        out_shape=jax.ShapeDtypeStruct((1, 128), jnp.float32),
        scratch_shapes=[pltpu.SMEM((1, 1), jnp.float32)],
        compiler_params=pltpu.CompilerParams(
            dimension_semantics=("arbitrary",),
        ),
    )(orig_embeds, masked_embeds, masks)
    return (out[0, 0], out[0, 1], out[0, 2])


# speedup vs baseline: 1.2241x; 1.1573x over previous
import jax
import jax.numpy as jnp
from jax.experimental import pallas as pl
from jax.experimental.pallas import tpu as pltpu

ALPHA = 20.0
BETA = 0.01

B = 128
D = 1024
T = 32768
BLOCK_T = 8192
NSTEPS = T // BLOCK_T


def _loss_kernel(orig_ref, masked_ref, masks_ref, out_ref, acc_ref):
    i = pl.program_id(0)

    @pl.when(i == 0)
    def _init():
        acc_ref[0, 0] = 0.0

    t = jnp.tanh(masks_ref[...] * 0.5)
    acc_ref[0, 0] += jnp.sum(t * t)

    @pl.when(i == NSTEPS - 1)
    def _finish():
        o = orig_ref[...]
        m = masked_ref[...]
        d = m - o
        mse = jnp.sum(d * d) * (1.0 / (B * D))
        na = jnp.maximum(jnp.sqrt(jnp.sum(o * o, axis=1)), 1e-8)
        nb = jnp.maximum(jnp.sqrt(jnp.sum(m * m, axis=1)), 1e-8)
        dot = jnp.sum(o * m, axis=1)
        cos = jnp.mean(1.0 - dot / (na * nb))
        semantic = mse + 0.1 * cos
        binary = 0.25 - 0.25 * acc_ref[0, 0] * (1.0 / (B * T))
        total = ALPHA * semantic + BETA * binary
        lane = jax.lax.broadcasted_iota(jnp.int32, (1, 128), 1)
        row = jnp.where(
            lane == 0, total,
            jnp.where(lane == 1, semantic,
                      jnp.where(lane == 2, binary, 0.0)))
        out_ref[...] = row


def kernel(orig_embeds, masked_embeds, masks):
    out = pl.pallas_call(
        _loss_kernel,
        grid=(NSTEPS,),
        in_specs=[
            pl.BlockSpec((B, D), lambda i: (0, 0)),
            pl.BlockSpec((B, D), lambda i: (0, 0)),
            pl.BlockSpec((B, BLOCK_T), lambda i: (0, i)),
        ],
        out_specs=pl.BlockSpec((1, 128), lambda i: (0, 0)),
        out_shape=jax.ShapeDtypeStruct((1, 128), jnp.float32),
        scratch_shapes=[pltpu.SMEM((1, 1), jnp.float32)],
        compiler_params=pltpu.CompilerParams(
            dimension_semantics=("arbitrary",),
        ),
    )(orig_embeds, masked_embeds, masks)
    return (out[0, 0], out[0, 1], out[0, 2])


# speedup vs baseline: 1.2272x; 1.0025x over previous
import jax
import jax.numpy as jnp
from jax.experimental import pallas as pl
from jax.experimental.pallas import tpu as pltpu

ALPHA = 20.0
BETA = 0.01

B = 128
D = 1024
T = 32768
BLOCK_T = 16384
NSTEPS = T // BLOCK_T


def _loss_kernel(orig_ref, masked_ref, masks_ref, out_ref, acc_ref):
    i = pl.program_id(0)

    @pl.when(i == 0)
    def _init():
        acc_ref[0, 0] = 0.0

    t = jnp.tanh(masks_ref[...] * 0.5)
    acc_ref[0, 0] += jnp.sum(t * t)

    @pl.when(i == NSTEPS - 1)
    def _finish():
        o = orig_ref[...]
        m = masked_ref[...]
        d = m - o
        mse = jnp.sum(d * d) * (1.0 / (B * D))
        na = jnp.maximum(jnp.sqrt(jnp.sum(o * o, axis=1)), 1e-8)
        nb = jnp.maximum(jnp.sqrt(jnp.sum(m * m, axis=1)), 1e-8)
        dot = jnp.sum(o * m, axis=1)
        cos = jnp.mean(1.0 - dot / (na * nb))
        semantic = mse + 0.1 * cos
        binary = 0.25 - 0.25 * acc_ref[0, 0] * (1.0 / (B * T))
        total = ALPHA * semantic + BETA * binary
        lane = jax.lax.broadcasted_iota(jnp.int32, (1, 128), 1)
        row = jnp.where(
            lane == 0, total,
            jnp.where(lane == 1, semantic,
                      jnp.where(lane == 2, binary, 0.0)))
        out_ref[...] = row


def kernel(orig_embeds, masked_embeds, masks):
    out = pl.pallas_call(
        _loss_kernel,
        grid=(NSTEPS,),
        in_specs=[
            pl.BlockSpec((B, D), lambda i: (0, 0)),
            pl.BlockSpec((B, D), lambda i: (0, 0)),
            pl.BlockSpec((B, BLOCK_T), lambda i: (0, i)),
        ],
        out_specs=pl.BlockSpec((1, 128), lambda i: (0, 0)),
        out_shape=jax.ShapeDtypeStruct((1, 128), jnp.float32),
        scratch_shapes=[pltpu.SMEM((1, 1), jnp.float32)],
        compiler_params=pltpu.CompilerParams(
            dimension_semantics=("arbitrary",),
        ),
    )(orig_embeds, masked_embeds, masks)
    return (out[0, 0], out[0, 1], out[0, 2])


# speedup vs baseline: 1.2600x; 1.0267x over previous
import jax
import jax.numpy as jnp
from jax.experimental import pallas as pl
from jax.experimental.pallas import tpu as pltpu

ALPHA = 20.0
BETA = 0.01

B = 128
D = 1024
T = 32768
BLOCK_T = 4096
NSTEPS = T // (2 * BLOCK_T)


def _loss_kernel(orig_ref, masked_ref, masks_lo_ref, masks_hi_ref, out_ref,
                 acc_ref):
    i = pl.program_id(0)

    @pl.when(i == 0)
    def _init():
        acc_ref[0, 0] = 0.0

    t0 = jnp.tanh(masks_lo_ref[...] * 0.5)
    t1 = jnp.tanh(masks_hi_ref[...] * 0.5)
    acc_ref[0, 0] += jnp.sum(t0 * t0) + jnp.sum(t1 * t1)

    @pl.when(i == NSTEPS - 1)
    def _finish():
        o = orig_ref[...]
        m = masked_ref[...]
        d = m - o
        mse = jnp.sum(d * d) * (1.0 / (B * D))
        na = jnp.maximum(jnp.sqrt(jnp.sum(o * o, axis=1)), 1e-8)
        nb = jnp.maximum(jnp.sqrt(jnp.sum(m * m, axis=1)), 1e-8)
        dot = jnp.sum(o * m, axis=1)
        cos = jnp.mean(1.0 - dot / (na * nb))
        semantic = mse + 0.1 * cos
        binary = 0.25 - 0.25 * acc_ref[0, 0] * (1.0 / (B * T))
        total = ALPHA * semantic + BETA * binary
        lane = jax.lax.broadcasted_iota(jnp.int32, (1, 128), 1)
        row = jnp.where(
            lane == 0, total,
            jnp.where(lane == 1, semantic,
                      jnp.where(lane == 2, binary, 0.0)))
        out_ref[...] = row


def kernel(orig_embeds, masked_embeds, masks):
    out = pl.pallas_call(
        _loss_kernel,
        grid=(NSTEPS,),
        in_specs=[
            pl.BlockSpec((B, D), lambda i: (0, 0)),
            pl.BlockSpec((B, D), lambda i: (0, 0)),
            pl.BlockSpec((B, BLOCK_T), lambda i: (0, i)),
            pl.BlockSpec((B, BLOCK_T), lambda i: (0, i + NSTEPS)),
        ],
        out_specs=pl.BlockSpec((1, 128), lambda i: (0, 0)),
        out_shape=jax.ShapeDtypeStruct((1, 128), jnp.float32),
        scratch_shapes=[pltpu.SMEM((1, 1), jnp.float32)],
        compiler_params=pltpu.CompilerParams(
            dimension_semantics=("arbitrary",),
        ),
    )(orig_embeds, masked_embeds, masks, masks)
    return (out[0, 0], out[0, 1], out[0, 2])
